# Initial kernel scaffold; baseline (speedup 1.0000x reference)
import jax
import jax.numpy as jnp
from jax.experimental import pallas as pl


def kernel(x, edge_index, W1l, b1l, W1r, W2l, b2l, W2r, gamma, beta, W3l, b3l, W3r):
    raise NotImplementedError("write your pallas kernel here")


# speedup vs baseline: 2.4040x; 2.4040x over previous
import functools

import jax
import jax.numpy as jnp
from jax import lax
from jax.experimental import pallas as pl
from jax.experimental.pallas import tpu as pltpu
from jax.experimental.pallas import tpu_sc as plsc

N = 10000
D = 128
E = 320000

NPAD = 10240
BLK = 1024
NBLK = NPAD // BLK

NC = 2
NS = 16
NW = NC * NS
CHUNK = 128
DW = D + 16
CPW = 80
GROUP = 8
NGRP = CPW // GROUP
EPAD = NW * CPW * CHUNK
ROWS_PER_S = NPAD // NS
RPS_CHUNKS = ROWS_PER_S // CHUNK

_f32 = jnp.float32


@functools.cache
def _build_sc_segsum(width):
    mesh = plsc.VectorSubcoreMesh(core_axis_name="c", subcore_axis_name="s")

    def _body(table_hbm, srcw_hbm, dstw_hbm, zeros_hbm, iota_hbm,
              out_parts,
              src_v, dst_v, iota_v, rows_v, acc_sh, sem):
        c = lax.axis_index("c")
        s = lax.axis_index("s")
        wid = c * NS + s
        rlo = s * ROWS_PER_S

        pltpu.sync_copy(zeros_hbm, rows_v)
        pltpu.sync_copy(iota_hbm.at[s], iota_v)

        def _init(i, carry):
            pltpu.sync_copy(rows_v, acc_sh.at[iota_v.at[i]])
            return carry

        lax.fori_loop(0, RPS_CHUNKS, _init, 0)
        plsc.subcore_barrier()

        def _grp(g, carry):
            pltpu.sync_copy(srcw_hbm.at[wid, pl.ds(g * GROUP, GROUP)], src_v)
            pltpu.sync_copy(dstw_hbm.at[wid, pl.ds(g * GROUP, GROUP)], dst_v)

            def _chunk(k, inner):
                pltpu.async_copy(table_hbm.at[src_v.at[k]], rows_v,
                                 sem).wait()
                pltpu.sync_copy(rows_v, acc_sh.at[dst_v.at[k]], add=True)
                return inner

            lax.fori_loop(0, GROUP, _chunk, 0)
            return carry

        lax.fori_loop(0, NGRP, _grp, 0)
        plsc.subcore_barrier()

        def _flush(i, carry):
            r = rlo + i * CHUNK
            pltpu.async_copy(acc_sh.at[iota_v.at[i]], rows_v, sem).wait()
            pltpu.sync_copy(rows_v, out_parts.at[c, pl.ds(r, CHUNK)])
            return carry

        lax.fori_loop(0, RPS_CHUNKS, _flush, 0)

    return pl.kernel(
        _body,
        mesh=mesh,
        out_type=jax.ShapeDtypeStruct((NC, NPAD, width), _f32),
        scratch_types=[
            pltpu.VMEM((GROUP, CHUNK), jnp.int32),
            pltpu.VMEM((GROUP, CHUNK), jnp.int32),
            pltpu.VMEM((RPS_CHUNKS, CHUNK), jnp.int32),
            pltpu.VMEM((CHUNK, width), _f32),
            pltpu.VMEM_SHARED((NPAD, width), _f32),
            pltpu.SemaphoreType.DMA,
        ],
    )


def _sc_segsum(table, src_p, dst_p, zeros, iota):
    return _build_sc_segsum(table.shape[1])(table, src_p, dst_p, zeros, iota)


@functools.cache
def _build_sc_counts():
    mesh = plsc.VectorSubcoreMesh(core_axis_name="c", subcore_axis_name="s")

    def _body(dstw_hbm, zeros_hbm, ones_hbm, iota_hbm,
              out_cnt,
              dst_v, iota_v, rows_v, acc_sh, sem):
        c = lax.axis_index("c")
        s = lax.axis_index("s")
        wid = c * NS + s
        rlo = s * ROWS_PER_S

        pltpu.sync_copy(zeros_hbm, rows_v)
        pltpu.sync_copy(iota_hbm.at[s], iota_v)

        def _init(i, carry):
            pltpu.sync_copy(rows_v, acc_sh.at[iota_v.at[i]])
            return carry

        lax.fori_loop(0, RPS_CHUNKS, _init, 0)
        pltpu.sync_copy(ones_hbm, rows_v)
        plsc.subcore_barrier()

        def _grp(g, carry):
            pltpu.sync_copy(dstw_hbm.at[wid, pl.ds(g * GROUP, GROUP)], dst_v)

            def _chunk(k, inner):
                pltpu.sync_copy(rows_v, acc_sh.at[dst_v.at[k]], add=True)
                return inner

            lax.fori_loop(0, GROUP, _chunk, 0)
            return carry

        lax.fori_loop(0, NGRP, _grp, 0)
        plsc.subcore_barrier()

        def _flush(i, carry):
            r = rlo + i * CHUNK
            pltpu.async_copy(acc_sh.at[iota_v.at[i]], rows_v, sem).wait()
            pltpu.sync_copy(rows_v, out_cnt.at[c, pl.ds(r, CHUNK)])
            return carry

        lax.fori_loop(0, RPS_CHUNKS, _flush, 0)

    return pl.kernel(
        _body,
        mesh=mesh,
        out_type=jax.ShapeDtypeStruct((NC, NPAD, D), _f32),
        scratch_types=[
            pltpu.VMEM((GROUP, CHUNK), jnp.int32),
            pltpu.VMEM((RPS_CHUNKS, CHUNK), jnp.int32),
            pltpu.VMEM((CHUNK, D), _f32),
            pltpu.VMEM_SHARED((NPAD, D), _f32),
            pltpu.SemaphoreType.DMA,
        ],
    )


def _conv_tail(agg, h_ref, wl_ref, wr_ref, b_ref):
    out = jnp.dot(agg, wl_ref[...], preferred_element_type=_f32)
    out += jnp.dot(h_ref[...], wr_ref[...], preferred_element_type=_f32)
    return out + b_ref[...]


def _k1_body(parts_ref, cntp_ref, x_ref, wl_ref, wr_ref, b_ref, o_ref,
             inv_ref):
    cnt = cntp_ref[0, :, 0] + cntp_ref[1, :, 0]
    inv = (1.0 / jnp.maximum(cnt, 1.0))[:, None]
    inv_ref[...] = inv
    agg = (parts_ref[0] + parts_ref[1]) * inv
    h = _conv_tail(agg, x_ref, wl_ref, wr_ref, b_ref)
    o_ref[...] = jnp.maximum(h, 0.0)


def _k2a_body(parts_ref, inv_ref, h_ref, wl_ref, wr_ref, b_ref,
              g_ref, s1_ref, s2_ref):
    agg = (parts_ref[0] + parts_ref[1]) * inv_ref[...]
    g = _conv_tail(agg, h_ref, wl_ref, wr_ref, b_ref)
    g_ref[...] = g
    row = lax.broadcasted_iota(jnp.int32, (BLK, 1), 0) + pl.program_id(0) * BLK
    gm = jnp.where(row < N, g, 0.0)
    s1_ref[...] = jnp.sum(gm, axis=0, keepdims=True)[None]
    s2_ref[...] = jnp.sum(gm * gm, axis=0, keepdims=True)[None]


def _k2b_body(g_ref, s1_ref, s2_ref, gamma_ref, beta_ref, x_ref, o_ref):
    mean = jnp.sum(s1_ref[...], axis=0) / N
    ex2 = jnp.sum(s2_ref[...], axis=0) / N
    var = ex2 - mean * mean
    h = (g_ref[...] - mean) * lax.rsqrt(var + 1e-5) * gamma_ref[...]
    h = h + beta_ref[...]
    o_ref[...] = jnp.maximum(h, 0.0) + x_ref[...]


def _k3_body(parts_ref, inv_ref, h_ref, wl_ref, wr_ref, b_ref, o_ref):
    agg = (parts_ref[0] + parts_ref[1]) * inv_ref[...]
    o = _conv_tail(agg, h_ref, wl_ref, wr_ref, b_ref)
    m = jnp.max(o, axis=1, keepdims=True)
    lse = jnp.log(jnp.sum(jnp.exp(o - m), axis=1, keepdims=True)) + m
    o_ref[...] = o - lse


_PARTS_SPEC = pl.BlockSpec((NC, BLK, D), lambda i: (0, i, 0))
_INV_SPEC = pl.BlockSpec((BLK, 1), lambda i: (i, 0))
_H_SPEC = pl.BlockSpec((BLK, D), lambda i: (i, 0))
_W_SPEC = pl.BlockSpec((D, D), lambda i: (0, 0))
_B_SPEC = pl.BlockSpec((1, D), lambda i: (0, 0))
_COL_SPEC = pl.BlockSpec((1, 1, D), lambda i: (i, 0, 0))
_COL_FULL_SPEC = pl.BlockSpec((NBLK, 1, D), lambda i: (0, 0, 0))

_HD = jax.ShapeDtypeStruct((NPAD, D), _f32)


def _k1(parts, cntp, x, wl, wr, b):
    return pl.pallas_call(
        _k1_body, grid=(NBLK,),
        in_specs=[_PARTS_SPEC, _PARTS_SPEC, _H_SPEC, _W_SPEC, _W_SPEC,
                  _B_SPEC],
        out_specs=[_H_SPEC, _INV_SPEC],
        out_shape=[_HD, jax.ShapeDtypeStruct((NPAD, 1), _f32)],
    )(parts, cntp, x, wl, wr, b)


def _k2a(parts, inv, h, wl, wr, b):
    return pl.pallas_call(
        _k2a_body, grid=(NBLK,),
        in_specs=[_PARTS_SPEC, _INV_SPEC, _H_SPEC, _W_SPEC, _W_SPEC, _B_SPEC],
        out_specs=[_H_SPEC, _COL_SPEC, _COL_SPEC],
        out_shape=[_HD,
                   jax.ShapeDtypeStruct((NBLK, 1, D), _f32),
                   jax.ShapeDtypeStruct((NBLK, 1, D), _f32)],
    )(parts, inv, h, wl, wr, b)


def _k2b(g, s1, s2, gamma, beta, x):
    return pl.pallas_call(
        _k2b_body, grid=(NBLK,),
        in_specs=[_H_SPEC, _COL_FULL_SPEC, _COL_FULL_SPEC,
                  _B_SPEC, _B_SPEC, _H_SPEC],
        out_specs=_H_SPEC, out_shape=_HD,
    )(g, s1, s2, gamma, beta, x)


def _k3(parts, inv, h, wl, wr, b):
    return pl.pallas_call(
        _k3_body, grid=(NBLK,),
        in_specs=[_PARTS_SPEC, _INV_SPEC, _H_SPEC, _W_SPEC, _W_SPEC, _B_SPEC],
        out_specs=_H_SPEC, out_shape=_HD,
    )(parts, inv, h, wl, wr, b)


def kernel(x, edge_index, W1l, b1l, W1r, W2l, b2l, W2r, gamma, beta,
           W3l, b3l, W3r):
    src = edge_index[0].astype(jnp.int32)
    dst = edge_index[1].astype(jnp.int32)

    x_pad = jnp.pad(x, ((0, NPAD - N), (0, 0)))
    src_p = jnp.concatenate(
        [src, jnp.zeros((EPAD - E,), jnp.int32)]).reshape(NW, CPW, CHUNK)
    dst_p = jnp.concatenate(
        [dst, jnp.full((EPAD - E,), N, jnp.int32)]).reshape(NW, CPW, CHUNK)
    zeros_d = jnp.zeros((CHUNK, D), _f32)
    ones_d = jnp.ones((CHUNK, D), _f32)
    iota = jnp.arange(NPAD, dtype=jnp.int32).reshape(NS, RPS_CHUNKS, CHUNK)

    w1l, w1r = W1l.T, W1r.T
    w2l, w2r = W2l.T, W2r.T
    w3l, w3r = W3l.T, W3r.T
    b1 = b1l.reshape(1, D)
    b2 = b2l.reshape(1, D)
    b3 = b3l.reshape(1, D)

    cntp = _build_sc_counts()(dst_p, zeros_d, ones_d, iota)
    parts1 = _sc_segsum(x_pad, src_p, dst_p, zeros_d, iota)
    h1, inv = _k1(parts1, cntp, x_pad, w1l, w1r, b1)

    parts2 = _sc_segsum(h1, src_p, dst_p, zeros_d, iota)
    g, s1, s2 = _k2a(parts2, inv, h1, w2l, w2r, b2)
    h2 = _k2b(g, s1, s2, gamma.reshape(1, D), beta.reshape(1, D), x_pad)

    parts3 = _sc_segsum(h2, src_p, dst_p, zeros_d, iota)
    o = _k3(parts3, inv, h2, w3l, w3r, b3)
    return o[:N]
